# baseline (speedup 1.0000x reference)
import functools

import jax
import jax.numpy as jnp
from jax import lax
from jax.experimental import pallas as pl
from jax.experimental.pallas import tpu as pltpu
from jax.experimental.pallas import tpu_sc as plsc

_N, _C = 100000, 80
_NW = 32
_RPW = 3136
_NP = _NW * _RPW

_TC_BLK = 10240
_TC_G = -(-_N // _TC_BLK)
_PPN = _TC_G * _TC_BLK


def _tc_body(p_ref, t_ref, sum_ref, pp_ref):
    i = pl.program_id(0)
    row_iota = lax.broadcasted_iota(jnp.int32, (_C, 128), 0)
    ones_c = jnp.ones((1, _C), jnp.float32)

    def chunk(k, acc):
        xt = p_ref[:, pl.ds(k * 128, 128)]
        th = jnp.tanh(xt * 0.5)
        sig = 0.5 + 0.5 * th
        sp = -jnp.log(0.5 - 0.5 * th)
        base = sp * sig * sig
        col = i * _TC_BLK + k * 128 + lax.broadcasted_iota(
            jnp.int32, (1, 128), 1)
        base = jnp.where(col < _N, base, 0.0)
        acc = acc + jnp.dot(ones_c, base,
                            preferred_element_type=jnp.float32)

        tgt = t_ref[pl.ds(k * 128, 128)]
        bc = (tgt - 1)[None, :]
        ohT = row_iota == bc
        ppc = jnp.dot(ones_c, jnp.where(ohT, xt, 0.0),
                      preferred_element_type=jnp.float32)
        pp_ref[pl.ds(k * 128, 128)] = ppc[0]
        return acc

    acc = lax.fori_loop(0, _TC_BLK // 128, chunk,
                        jnp.zeros((1, 128), jnp.float32), unroll=8)

    @pl.when(i == 0)
    def _():
        sum_ref[...] = jnp.zeros((128,), jnp.float32)

    sum_ref[...] += acc[0]


_tc_dense = pl.pallas_call(
    _tc_body,
    grid=(_TC_G,),
    in_specs=[
        pl.BlockSpec((_C, _TC_BLK), lambda i: (0, i)),
        pl.BlockSpec((_TC_BLK,), lambda i: (i,)),
    ],
    out_specs=[
        pl.BlockSpec((128,), lambda i: (0,)),
        pl.BlockSpec((_TC_BLK,), lambda i: (i,)),
    ],
    out_shape=[
        jax.ShapeDtypeStruct((128,), jnp.float32),
        jax.ShapeDtypeStruct((_PPN,), jnp.float32),
    ],
)


def _sc_mesh():
    return plsc.VectorSubcoreMesh(core_axis_name="c", subcore_axis_name="s")


@functools.partial(
    pl.kernel,
    mesh=_sc_mesh(),
    out_type=jax.ShapeDtypeStruct((_NW, 16), jnp.float32),
    scratch_types=[
        pltpu.VMEM((_RPW,), jnp.float32),
        pltpu.VMEM((_RPW,), jnp.int32),
        pltpu.VMEM((_RPW,), jnp.float32),
        pltpu.VMEM((16,), jnp.float32),
        pltpu.VMEM((128,), jnp.float32),
    ],
)
def _sc_corr(sum_hbm, pp_hbm, tgt_hbm, scr_hbm, out_hbm,
             pp_v, tgt_v, scr_v, acc_v, s_v):
    cid = lax.axis_index("c")
    sid = lax.axis_index("s")
    wid = sid * 2 + cid
    first = wid * _RPW
    row0 = jnp.minimum(first, _N - _RPW)
    pltpu.sync_copy(pp_hbm.at[pl.ds(row0, _RPW)], pp_v)
    pltpu.sync_copy(tgt_hbm.at[pl.ds(row0, _RPW)], tgt_v)
    pltpu.sync_copy(scr_hbm.at[pl.ds(row0, _RPW)], scr_v)
    lane = lax.broadcasted_iota(jnp.int32, (16,), 0)

    def accumulate(i, acc):
        off = i * 16
        tgt = tgt_v[pl.ds(off, 16)]
        s = scr_v[pl.ds(off, 16)]
        x = pp_v[pl.ds(off, 16)]
        pos = ((tgt - 1) >= 0) & ((row0 + off + lane) >= first)
        t = jnp.exp(-jnp.abs(x))
        v = t / (2.0 + t)
        v2 = v * v
        poly = 1.0 + v2 * (1.0 / 3.0 + v2 * (1.0 / 5.0 + v2 * (
            1.0 / 7.0 + v2 * (1.0 / 9.0 + v2 * (1.0 / 11.0 + v2 * (1.0 / 13.0))))))
        ell = 2.0 * v * poly
        sig = jnp.where(x >= 0.0, 1.0, t) / (1.0 + t)
        b0 = jnp.maximum(x, 0.0) + ell
        base = b0 * sig * sig
        pt = s - sig
        new = (b0 - x * s) * pt * pt
        return acc + jnp.where(pos, new - base, 0.0)

    acc = lax.fori_loop(0, _RPW // 16, accumulate,
                    jnp.zeros((16,), jnp.float32), unroll=4)
    acc_v[...] = acc

    @pl.when(wid == 0)
    def _():
        pltpu.sync_copy(sum_hbm, s_v)
        for k in range(8):
            acc_v[...] += s_v[pl.ds(k * 16, 16)]

    pltpu.sync_copy(acc_v, out_hbm.at[wid])


def kernel(pred, target, score):
    tgt = target.astype(jnp.int32)
    scr = score.astype(jnp.float32)

    sum_l, pp_p = _tc_dense(pred.T, tgt)
    parts = _sc_corr(sum_l, pp_p, tgt, scr)

    return jnp.sum(parts) / jnp.float32(_N * _C)


# speedup vs baseline: 1.1862x; 1.1862x over previous
import functools

import jax
import jax.numpy as jnp
from jax import lax
from jax.experimental import pallas as pl
from jax.experimental.pallas import tpu as pltpu
from jax.experimental.pallas import tpu_sc as plsc

_N, _C = 100000, 80
_NW = 32
_RPW = 3136
_NP = _NW * _RPW

_TC_BLK = 10240
_TC_G = -(-_N // _TC_BLK)
_PPN = _TC_G * _TC_BLK


def _tc_body(p_ref, t_ref, sum_ref, pp_ref):
    i = pl.program_id(0)
    row_iota1 = lax.broadcasted_iota(jnp.int32, (_C, 128), 0) + 1

    def make_chunk(masked):
        def chunk(k, acc):
            xt = p_ref[:, pl.ds(k * 128, 128)]
            th = jnp.tanh(xt * 0.5)
            sig = 0.5 + 0.5 * th
            sp = -jnp.log(0.5 - 0.5 * th)
            base = sp * sig * sig
            if masked:
                col = i * _TC_BLK + k * 128 + lax.broadcasted_iota(
                    jnp.int32, (1, 128), 1)
                base = jnp.where(col < _N, base, 0.0)
            acc = acc + jnp.sum(base.reshape(_C // 8, 8, 128), axis=0)

            tgt = t_ref[pl.ds(k * 128, 128)]
            ohT = row_iota1 == tgt[None, :]
            pp_ref[pl.ds(k * 128, 128)] = jnp.sum(jnp.where(ohT, xt, 0.0),
                                                  axis=0)
            return acc

        return chunk

    acc0 = jnp.zeros((8, 128), jnp.float32)
    nk = _TC_BLK // 128

    @pl.when(i < _TC_G - 1)
    def _():
        acc = lax.fori_loop(0, nk, make_chunk(False), acc0, unroll=8)

        @pl.when(i == 0)
        def _():
            sum_ref[...] = jnp.zeros((128,), jnp.float32)

        sum_ref[...] += jnp.sum(acc, axis=0)

    @pl.when(i == _TC_G - 1)
    def _():
        acc = lax.fori_loop(0, nk, make_chunk(True), acc0, unroll=8)
        sum_ref[...] += jnp.sum(acc, axis=0)


_tc_dense = pl.pallas_call(
    _tc_body,
    grid=(_TC_G,),
    in_specs=[
        pl.BlockSpec((_C, _TC_BLK), lambda i: (0, i)),
        pl.BlockSpec((_TC_BLK,), lambda i: (i,)),
    ],
    out_specs=[
        pl.BlockSpec((128,), lambda i: (0,)),
        pl.BlockSpec((_TC_BLK,), lambda i: (i,)),
    ],
    out_shape=[
        jax.ShapeDtypeStruct((128,), jnp.float32),
        jax.ShapeDtypeStruct((_PPN,), jnp.float32),
    ],
)


def _sc_mesh():
    return plsc.VectorSubcoreMesh(core_axis_name="c", subcore_axis_name="s")


@functools.partial(
    pl.kernel,
    mesh=_sc_mesh(),
    out_type=jax.ShapeDtypeStruct((_NW, 16), jnp.float32),
    scratch_types=[
        pltpu.VMEM((_RPW,), jnp.float32),
        pltpu.VMEM((_RPW,), jnp.int32),
        pltpu.VMEM((_RPW,), jnp.float32),
        pltpu.VMEM((16,), jnp.float32),
        pltpu.VMEM((128,), jnp.float32),
    ],
)
def _sc_corr(sum_hbm, pp_hbm, tgt_hbm, scr_hbm, out_hbm,
             pp_v, tgt_v, scr_v, acc_v, s_v):
    cid = lax.axis_index("c")
    sid = lax.axis_index("s")
    wid = sid * 2 + cid
    first = wid * _RPW
    row0 = jnp.minimum(first, _N - _RPW)
    pltpu.sync_copy(pp_hbm.at[pl.ds(row0, _RPW)], pp_v)
    pltpu.sync_copy(tgt_hbm.at[pl.ds(row0, _RPW)], tgt_v)
    pltpu.sync_copy(scr_hbm.at[pl.ds(row0, _RPW)], scr_v)
    lane = lax.broadcasted_iota(jnp.int32, (16,), 0)

    def accumulate(i, acc):
        off = i * 16
        tgt = tgt_v[pl.ds(off, 16)]
        s = scr_v[pl.ds(off, 16)]
        x = pp_v[pl.ds(off, 16)]
        pos = ((tgt - 1) >= 0) & ((row0 + off + lane) >= first)
        t = jnp.exp(-jnp.abs(x))
        v = t / (2.0 + t)
        v2 = v * v
        poly = 1.0 + v2 * (1.0 / 3.0 + v2 * (1.0 / 5.0 + v2 * (
            1.0 / 7.0 + v2 * (1.0 / 9.0 + v2 * (1.0 / 11.0 + v2 * (1.0 / 13.0))))))
        ell = 2.0 * v * poly
        sig = jnp.where(x >= 0.0, 1.0, t) / (1.0 + t)
        b0 = jnp.maximum(x, 0.0) + ell
        base = b0 * sig * sig
        pt = s - sig
        new = (b0 - x * s) * pt * pt
        return acc + jnp.where(pos, new - base, 0.0)

    acc = lax.fori_loop(0, _RPW // 16, accumulate,
                    jnp.zeros((16,), jnp.float32), unroll=4)
    acc_v[...] = acc

    @pl.when(wid == 0)
    def _():
        pltpu.sync_copy(sum_hbm, s_v)
        for k in range(8):
            acc_v[...] += s_v[pl.ds(k * 16, 16)]

    pltpu.sync_copy(acc_v, out_hbm.at[wid])


def kernel(pred, target, score):
    tgt = target.astype(jnp.int32)
    scr = score.astype(jnp.float32)

    sum_l, pp_p = _tc_dense(pred.T, tgt)
    parts = _sc_corr(sum_l, pp_p, tgt, scr)

    return jnp.sum(parts) / jnp.float32(_N * _C)


# speedup vs baseline: 1.1936x; 1.0062x over previous
import functools

import jax
import jax.numpy as jnp
from jax import lax
from jax.experimental import pallas as pl
from jax.experimental.pallas import tpu as pltpu
from jax.experimental.pallas import tpu_sc as plsc

_N, _C = 100000, 80
_NW = 32
_RPW = 3136
_NP = _NW * _RPW

_TC_BLK = 20480
_TC_G = -(-_N // _TC_BLK)
_PPN = _TC_G * _TC_BLK


def _tc_body(p_ref, t_ref, sum_ref, pp_ref):
    i = pl.program_id(0)
    row_iota1 = lax.broadcasted_iota(jnp.int32, (_C, 128), 0) + 1

    def make_chunk(masked):
        def chunk(k, acc):
            xt = p_ref[:, pl.ds(k * 128, 128)]
            th = jnp.tanh(xt * 0.5)
            sig = 0.5 + 0.5 * th
            sp = -jnp.log(0.5 - 0.5 * th)
            base = sp * sig * sig
            if masked:
                col = i * _TC_BLK + k * 128 + lax.broadcasted_iota(
                    jnp.int32, (1, 128), 1)
                base = jnp.where(col < _N, base, 0.0)
            acc = acc + jnp.sum(base.reshape(_C // 8, 8, 128), axis=0)

            tgt = t_ref[pl.ds(k * 128, 128)]
            ohT = row_iota1 == tgt[None, :]
            pp_ref[pl.ds(k * 128, 128)] = jnp.sum(jnp.where(ohT, xt, 0.0),
                                                  axis=0)
            return acc

        return chunk

    acc0 = jnp.zeros((8, 128), jnp.float32)
    nk = _TC_BLK // 128

    @pl.when(i < _TC_G - 1)
    def _():
        acc = lax.fori_loop(0, nk, make_chunk(False), acc0, unroll=8)

        @pl.when(i == 0)
        def _():
            sum_ref[...] = jnp.zeros((128,), jnp.float32)

        sum_ref[...] += jnp.sum(acc, axis=0)

    @pl.when(i == _TC_G - 1)
    def _():
        acc = lax.fori_loop(0, nk, make_chunk(True), acc0, unroll=8)
        sum_ref[...] += jnp.sum(acc, axis=0)


_tc_dense = pl.pallas_call(
    _tc_body,
    grid=(_TC_G,),
    in_specs=[
        pl.BlockSpec((_C, _TC_BLK), lambda i: (0, i)),
        pl.BlockSpec((_TC_BLK,), lambda i: (i,)),
    ],
    out_specs=[
        pl.BlockSpec((128,), lambda i: (0,)),
        pl.BlockSpec((_TC_BLK,), lambda i: (i,)),
    ],
    out_shape=[
        jax.ShapeDtypeStruct((128,), jnp.float32),
        jax.ShapeDtypeStruct((_PPN,), jnp.float32),
    ],
)


def _sc_mesh():
    return plsc.VectorSubcoreMesh(core_axis_name="c", subcore_axis_name="s")


@functools.partial(
    pl.kernel,
    mesh=_sc_mesh(),
    out_type=jax.ShapeDtypeStruct((_NW, 16), jnp.float32),
    scratch_types=[
        pltpu.VMEM((_RPW,), jnp.float32),
        pltpu.VMEM((_RPW,), jnp.int32),
        pltpu.VMEM((_RPW,), jnp.float32),
        pltpu.VMEM((16,), jnp.float32),
        pltpu.VMEM((128,), jnp.float32),
    ],
)
def _sc_corr(sum_hbm, pp_hbm, tgt_hbm, scr_hbm, out_hbm,
             pp_v, tgt_v, scr_v, acc_v, s_v):
    cid = lax.axis_index("c")
    sid = lax.axis_index("s")
    wid = sid * 2 + cid
    first = wid * _RPW
    row0 = jnp.minimum(first, _N - _RPW)
    pltpu.sync_copy(pp_hbm.at[pl.ds(row0, _RPW)], pp_v)
    pltpu.sync_copy(tgt_hbm.at[pl.ds(row0, _RPW)], tgt_v)
    pltpu.sync_copy(scr_hbm.at[pl.ds(row0, _RPW)], scr_v)
    lane = lax.broadcasted_iota(jnp.int32, (16,), 0)

    def accumulate(i, acc):
        off = i * 16
        tgt = tgt_v[pl.ds(off, 16)]
        s = scr_v[pl.ds(off, 16)]
        x = pp_v[pl.ds(off, 16)]
        pos = ((tgt - 1) >= 0) & ((row0 + off + lane) >= first)
        t = jnp.exp(-jnp.abs(x))
        v = t / (2.0 + t)
        v2 = v * v
        poly = 1.0 + v2 * (1.0 / 3.0 + v2 * (1.0 / 5.0 + v2 * (
            1.0 / 7.0 + v2 * (1.0 / 9.0 + v2 * (1.0 / 11.0 + v2 * (1.0 / 13.0))))))
        ell = 2.0 * v * poly
        sig = jnp.where(x >= 0.0, 1.0, t) / (1.0 + t)
        b0 = jnp.maximum(x, 0.0) + ell
        base = b0 * sig * sig
        pt = s - sig
        new = (b0 - x * s) * pt * pt
        return acc + jnp.where(pos, new - base, 0.0)

    acc = lax.fori_loop(0, _RPW // 16, accumulate,
                    jnp.zeros((16,), jnp.float32), unroll=4)
    acc_v[...] = acc

    @pl.when(wid == 0)
    def _():
        pltpu.sync_copy(sum_hbm, s_v)
        for k in range(8):
            acc_v[...] += s_v[pl.ds(k * 16, 16)]

    pltpu.sync_copy(acc_v, out_hbm.at[wid])


def kernel(pred, target, score):
    tgt = target.astype(jnp.int32)
    scr = score.astype(jnp.float32)

    sum_l, pp_p = _tc_dense(pred.T, tgt)
    parts = _sc_corr(sum_l, pp_p, tgt, scr)

    return jnp.sum(parts) / jnp.float32(_N * _C)
